# baseline (speedup 1.0000x reference)
import jax
import jax.numpy as jnp
from jax import lax
from jax.experimental import pallas as pl
from jax.experimental.pallas import tpu as pltpu
from jax.experimental.pallas import tpu_sc as plsc

N = 10000
E = 320000
D = 128
NP = 10240
NC = 2
NS = 16
NW = NC * NS
EW = E // NW
SUB = 80
JR = EW // SUB
SL = NP // NS
NPV = 10112
SLV = NPV // NS
VPS = SUB // 16
NEG = -3.0e38

_mesh = plsc.VectorSubcoreMesh(
    core_axis_name="c", subcore_axis_name="s", num_cores=NC, num_subcores=NS)


def _lrelu(x):
    return jnp.maximum(x, 0.2 * x)


def _ka_ff_body(f_ref, w_ref, a_ref, ff_ref):
    sft = jnp.dot(f_ref[...], w_ref[...], preferred_element_type=jnp.float32)
    ff_ref[...] = jnp.dot(sft, a_ref[...], preferred_element_type=jnp.float32)


_ka_ff = pl.pallas_call(
    _ka_ff_body,
    grid=(50,),
    in_specs=[
        pl.BlockSpec((200, D), lambda i: (i, 0)),
        pl.BlockSpec((D, D), lambda i: (0, 0)),
        pl.BlockSpec((D, D), lambda i: (0, 0)),
    ],
    out_specs=pl.BlockSpec((200, D), lambda i: (i, 0)),
    out_shape=jax.ShapeDtypeStruct((N, D), jnp.float32),
)


def _ka_seq_body(f_ref, w_ref, seq_ref):
    seq_ref[...] = jnp.dot(f_ref[...], w_ref[...],
                           preferred_element_type=jnp.float32)


_ka_seq = pl.pallas_call(
    _ka_seq_body,
    grid=(50,),
    in_specs=[
        pl.BlockSpec((200, D), lambda i: (i, 0)),
        pl.BlockSpec((D, D), lambda i: (0, 0)),
    ],
    out_specs=pl.BlockSpec((200, D), lambda i: (i, 0)),
    out_shape=jax.ShapeDtypeStruct((N, D), jnp.float32),
)


def _kb_body(row_hbm, col_hbm, f2_hbm, g_out,
             f2_l, g_l, rbuf, cbuf, tmp, acc, shared):
    c = lax.axis_index("c")
    s = lax.axis_index("s")
    wid = s * NC + c
    pltpu.sync_copy(f2_hbm, f2_l)
    pltpu.sync_copy(row_hbm.at[pl.ds(wid * EW, EW)], rbuf)
    pltpu.sync_copy(col_hbm.at[pl.ds(wid * EW, EW)], cbuf)

    def _init(i, carry):
        g_l[pl.ds(i * 16, 16)] = jnp.full((16,), NEG, jnp.float32)
        return carry
    lax.fori_loop(0, NP // 16, _init, 0)

    def _vec(v, carry):
        sl = pl.ds(v * 16, 16)
        rows = rbuf[sl]
        cols = cbuf[sl]
        vals = plsc.load_gather(f2_l, [cols])

        def _cond(active):
            return jnp.any(active)

        def _body(active):
            cur = plsc.load_gather(g_l, [rows])
            need = jnp.logical_and(active, vals > cur)
            plsc.store_scatter(g_l, [rows], vals, mask=need)
            chk = plsc.load_gather(g_l, [rows])
            return jnp.logical_and(need, chk < vals)

        lax.while_loop(_cond, _body, jnp.ones((16,), jnp.bool_))
        return carry
    lax.fori_loop(0, EW // 16, _vec, 0)

    pltpu.sync_copy(g_l, shared.at[s])
    plsc.subcore_barrier()
    base = s * SL
    pltpu.sync_copy(shared.at[0, pl.ds(base, SL)], acc)
    for t in range(1, NS):
        pltpu.sync_copy(shared.at[t, pl.ds(base, SL)], tmp)

        def _mx(v, carry):
            sl = pl.ds(v * 16, 16)
            acc[sl] = jnp.maximum(acc[sl], tmp[sl])
            return carry
        lax.fori_loop(0, SL // 16, _mx, 0)
    pltpu.sync_copy(acc, g_out.at[pl.ds(c * NP + base, SL)])


_kb = pl.kernel(
    _kb_body,
    out_type=jax.ShapeDtypeStruct((NC * NP,), jnp.float32),
    mesh=_mesh,
    compiler_params=pltpu.CompilerParams(needs_layout_passes=False),
    scratch_types=[
        pltpu.VMEM((NP,), jnp.float32),
        pltpu.VMEM((NP,), jnp.float32),
        pltpu.VMEM((EW,), jnp.int32),
        pltpu.VMEM((EW,), jnp.int32),
        pltpu.VMEM((SL,), jnp.float32),
        pltpu.VMEM((SL,), jnp.float32),
        pltpu.VMEM_SHARED((NS, NP), jnp.float32),
    ],
)


def _kc_body(row3_hbm, col_hbm, f1_hbm, f2_hbm, gp_hbm, den_out, ex_out,
             f1_l, f2_l, m_l, gb, cbuf, rstage, exb, exfull, zb, den_sh):
    c = lax.axis_index("c")
    s = lax.axis_index("s")
    wid = s * NC + c
    pltpu.sync_copy(f1_hbm, f1_l)
    pltpu.sync_copy(f2_hbm, f2_l)
    pltpu.sync_copy(col_hbm.at[pl.ds(wid * EW, EW)], cbuf)
    pltpu.sync_copy(row3_hbm.at[wid], rstage)
    pltpu.sync_copy(gp_hbm.at[pl.ds(0, NP)], m_l)
    pltpu.sync_copy(gp_hbm.at[pl.ds(NP, NP)], gb)

    def _m(i, carry):
        sl = pl.ds(i * 16, 16)
        m_l[sl] = _lrelu(f1_l[sl] + jnp.maximum(m_l[sl], gb[sl]))
        return carry
    lax.fori_loop(0, NP // 16, _m, 0)

    def _z(i, carry):
        zb[pl.ds(i * 16, 16)] = jnp.zeros((16,), jnp.float32)
        return carry
    lax.fori_loop(0, SL // 16, _z, 0)
    pltpu.sync_copy(zb, den_sh.at[pl.ds(s * SL, SL)])
    plsc.subcore_barrier()

    def _row(j, carry):
        for v in range(VPS):
            sl = pl.ds(v * 16, 16)
            rows = rstage[j, sl]
            cols = cbuf[pl.ds(j * SUB + v * 16, 16)]
            x = plsc.load_gather(f1_l, [rows]) + plsc.load_gather(f2_l, [cols])
            mv = plsc.load_gather(m_l, [rows])
            ev = jnp.exp(_lrelu(x) - mv)
            exb[sl] = ev
            exfull[pl.ds(j * SUB + v * 16, 16)] = ev
        pltpu.sync_copy(exb, den_sh.at[rstage.at[j]], add=True)
        return carry
    lax.fori_loop(0, JR, _row, 0)
    pltpu.sync_copy(exfull, ex_out.at[pl.ds(wid * EW, EW)])
    plsc.subcore_barrier()
    sl = pl.ds(s * SL, SL)
    pltpu.sync_copy(den_sh.at[sl], den_out.at[pl.ds(c * NP + s * SL, SL)])


_kc = pl.kernel(
    _kc_body,
    out_type=(jax.ShapeDtypeStruct((NC * NP,), jnp.float32),
              jax.ShapeDtypeStruct((E,), jnp.float32)),
    mesh=_mesh,
    compiler_params=pltpu.CompilerParams(needs_layout_passes=False),
    scratch_types=[
        pltpu.VMEM((NP,), jnp.float32),
        pltpu.VMEM((NP,), jnp.float32),
        pltpu.VMEM((NP,), jnp.float32),
        pltpu.VMEM((NP,), jnp.float32),
        pltpu.VMEM((EW,), jnp.int32),
        pltpu.VMEM((JR, SUB), jnp.int32),
        pltpu.VMEM((SUB,), jnp.float32),
        pltpu.VMEM((EW,), jnp.float32),
        pltpu.VMEM((SL,), jnp.float32),
        pltpu.VMEM_SHARED((NP,), jnp.float32),
    ],
)


def _kd_body(row3_hbm, col_hbm, seq_hbm, ex_hbm, dp_hbm, vals_out,
             den_l, db, cbuf, rst, exb, fb, vals_sh,
             semg0, semg1, semx, semr, semsc0, semsc1):
    c = lax.axis_index("c")
    s = lax.axis_index("s")
    wid = s * NC + c
    pltpu.sync_copy(dp_hbm.at[pl.ds(0, N)], den_l)
    pltpu.sync_copy(col_hbm.at[pl.ds(wid * EW, EW)], cbuf)
    for k in range(5):
        pltpu.sync_copy(dp_hbm.at[pl.ds(NP + k * 2000, 2000)], db)

        def _add(i, carry):
            sl = pl.ds(k * 2000 + i * 16, 16)
            den_l[sl] = den_l[sl] + db[pl.ds(i * 16, 16)]
            return carry
        lax.fori_loop(0, 125, _add, 0)

    def _zf(e, carry):
        for v in range(D // 16):
            fb[0, e, pl.ds(v * 16, 16)] = jnp.zeros((16,), jnp.float32)
        return carry
    lax.fori_loop(0, SUB, _zf, 0)
    for k in range(SLV // SUB):
        pltpu.sync_copy(fb.at[0], vals_sh.at[pl.ds(s * SLV + k * SUB, SUB)])
    pltpu.sync_copy(fb.at[0, pl.ds(0, SLV % SUB)],
                    vals_sh.at[pl.ds(s * SLV + (SLV // SUB) * SUB,
                                     SLV % SUB)])

    pltpu.sync_copy(row3_hbm.at[wid, 0], rst.at[0])
    pltpu.sync_copy(row3_hbm.at[wid, 1], rst.at[1])
    pltpu.sync_copy(ex_hbm.at[pl.ds(wid * EW, SUB)], exb)
    pltpu.async_copy(seq_hbm.at[cbuf.at[pl.ds(0, SUB)]], fb.at[0], semg0)
    plsc.subcore_barrier()

    semg = (semg0, semg1)
    semsc = (semsc0, semsc1)

    def _coefs(b):
        cos = []
        for v in range(VPS):
            sl = pl.ds(v * 16, 16)
            dv = plsc.load_gather(den_l, [rst[b, sl]])
            cos.append(exb[sl] / dv)
        return cos

    def _scale(b, cos):
        for g in range(VPS):
            cv = cos[g]
            for l in range(16):
                cf = cv[l]
                e = g * 16 + l
                for v in range(D // 16):
                    fsl = pl.ds(v * 16, 16)
                    fb[b, e, fsl] = fb[b, e, fsl] * cf

    def _part(j, b, first):
        b1 = 1 - b
        cpr = None
        if not first:
            pltpu.make_async_copy(seq_hbm.at[pl.ds(0, SUB)], fb.at[b1],
                                  semsc[b1]).wait()
            cpr = pltpu.async_copy(row3_hbm.at[wid, j + 1], rst.at[b1],
                                   semr)
        pltpu.async_copy(seq_hbm.at[cbuf.at[pl.ds((j + 1) * SUB, SUB)]],
                         fb.at[b1], semg[b1])
        cos = _coefs(b)
        cpx = pltpu.async_copy(
            ex_hbm.at[pl.ds(wid * EW + (j + 1) * SUB, SUB)], exb, semx)
        pltpu.make_async_copy(seq_hbm.at[pl.ds(0, SUB)], fb.at[b],
                              semg[b]).wait()
        _scale(b, cos)
        pltpu.async_copy(fb.at[b], vals_sh.at[rst.at[b]], semsc[b],
                         add=True)
        cpx.wait()
        if cpr is not None:
            cpr.wait()

    _part(0, 0, True)
    _part(1, 1, False)

    def _dbl(t, carry):
        _part(2 * t, 0, False)
        _part(2 * t + 1, 1, False)
        return carry
    lax.fori_loop(1, JR // 2, _dbl, 0)

    pltpu.make_async_copy(seq_hbm.at[pl.ds(0, SUB)], fb.at[1],
                          semsc[1]).wait()
    cos = _coefs(0)
    pltpu.make_async_copy(seq_hbm.at[pl.ds(0, SUB)], fb.at[0], semg0).wait()
    _scale(0, cos)
    pltpu.sync_copy(fb.at[0], vals_sh.at[rst.at[0]], add=True)

    plsc.subcore_barrier()
    sl2 = pl.ds(s * SLV, SLV)
    pltpu.sync_copy(vals_sh.at[sl2], vals_out.at[c, sl2])


_kd = pl.kernel(
    _kd_body,
    out_type=jax.ShapeDtypeStruct((NC, NPV, D), jnp.float32),
    mesh=_mesh,
    compiler_params=pltpu.CompilerParams(needs_layout_passes=False),
    scratch_types=[
        pltpu.VMEM((N,), jnp.float32),
        pltpu.VMEM((2000,), jnp.float32),
        pltpu.VMEM((EW,), jnp.int32),
        pltpu.VMEM((2, SUB), jnp.int32),
        pltpu.VMEM((SUB,), jnp.float32),
        pltpu.VMEM((2, SUB, D), jnp.float32),
        pltpu.VMEM_SHARED((NPV, D), jnp.float32),
        pltpu.SemaphoreType.DMA,
        pltpu.SemaphoreType.DMA,
        pltpu.SemaphoreType.DMA,
        pltpu.SemaphoreType.DMA,
        pltpu.SemaphoreType.DMA,
        pltpu.SemaphoreType.DMA,
    ],
)


def _ke_body(v0_ref, v1_ref, b_ref, o_ref):
    o_ref[...] = v0_ref[0] + v1_ref[0] + b_ref[...]


_ke = pl.pallas_call(
    _ke_body,
    grid=(50,),
    in_specs=[
        pl.BlockSpec((1, 200, D), lambda i: (0, i, 0)),
        pl.BlockSpec((1, 200, D), lambda i: (1, i, 0)),
        pl.BlockSpec((1, D), lambda i: (0, 0)),
    ],
    out_specs=pl.BlockSpec((200, D), lambda i: (i, 0)),
    out_shape=jax.ShapeDtypeStruct((N, D), jnp.float32),
)


def kernel(feat, edge_index, W, a_l_w, a_l_b, a_r_w, a_r_b, bias):
    a_cat = jnp.concatenate(
        [a_l_w, a_r_w, jnp.zeros((D, D - 2), jnp.float32)], axis=1)
    ff = _ka_ff(feat, W, a_cat)
    seq = _ka_seq(feat, W)
    f1 = jnp.pad(ff[:, 0] + a_l_b[0], (0, NP - N))
    f2 = jnp.pad(ff[:, 1] + a_r_b[0], (0, NP - N))
    row = edge_index[0]
    col = edge_index[1]
    row3 = row.reshape(NW, JR, SUB)
    g_p = _kb(row, col, f2)
    den_p, ex = _kc(row3, col, f1, f2, g_p)
    vals_p = _kd(row3, col, seq, ex, den_p)
    return _ke(vals_p, vals_p, bias.reshape(1, D))


# speedup vs baseline: 1.0038x; 1.0038x over previous
import jax
import jax.numpy as jnp
from jax import lax
from jax.experimental import pallas as pl
from jax.experimental.pallas import tpu as pltpu
from jax.experimental.pallas import tpu_sc as plsc

N = 10000
E = 320000
D = 128
NP = 10240
NC = 2
NS = 16
NW = NC * NS
EW = E // NW
SUB = 80
JR = EW // SUB
SL = NP // NS
NPV = 10112
SLV = NPV // NS
VPS = SUB // 16
NEG = -3.0e38

_mesh = plsc.VectorSubcoreMesh(
    core_axis_name="c", subcore_axis_name="s", num_cores=NC, num_subcores=NS)


def _lrelu(x):
    return jnp.maximum(x, 0.2 * x)


def _ka_body(f_ref, w_ref, a_ref, seq_ref, ff_ref):
    sft = jnp.dot(f_ref[...], w_ref[...], preferred_element_type=jnp.float32)
    seq_ref[...] = sft
    ff_ref[...] = jnp.dot(sft, a_ref[...], preferred_element_type=jnp.float32)


_ka = pl.pallas_call(
    _ka_body,
    grid=(50,),
    in_specs=[
        pl.BlockSpec((200, D), lambda i: (i, 0)),
        pl.BlockSpec((D, D), lambda i: (0, 0)),
        pl.BlockSpec((D, D), lambda i: (0, 0)),
    ],
    out_specs=[
        pl.BlockSpec((200, D), lambda i: (i, 0)),
        pl.BlockSpec((200, D), lambda i: (i, 0)),
    ],
    out_shape=[
        jax.ShapeDtypeStruct((N, D), jnp.float32),
        jax.ShapeDtypeStruct((N, D), jnp.float32),
    ],
)


def _kb_body(row_hbm, col_hbm, f2_hbm, g_out,
             f2_l, g_l, rbuf, cbuf, tmp, acc, shared):
    c = lax.axis_index("c")
    s = lax.axis_index("s")
    wid = s * NC + c
    pltpu.sync_copy(f2_hbm, f2_l)
    pltpu.sync_copy(row_hbm.at[pl.ds(wid * EW, EW)], rbuf)
    pltpu.sync_copy(col_hbm.at[pl.ds(wid * EW, EW)], cbuf)

    def _init(i, carry):
        g_l[pl.ds(i * 16, 16)] = jnp.full((16,), NEG, jnp.float32)
        return carry
    lax.fori_loop(0, NP // 16, _init, 0)

    def _vec(v, carry):
        sl = pl.ds(v * 16, 16)
        rows = rbuf[sl]
        cols = cbuf[sl]
        vals = plsc.load_gather(f2_l, [cols])

        def _cond(active):
            return jnp.any(active)

        def _body(active):
            cur = plsc.load_gather(g_l, [rows])
            need = jnp.logical_and(active, vals > cur)
            plsc.store_scatter(g_l, [rows], vals, mask=need)
            chk = plsc.load_gather(g_l, [rows])
            return jnp.logical_and(need, chk < vals)

        lax.while_loop(_cond, _body, jnp.ones((16,), jnp.bool_))
        return carry
    lax.fori_loop(0, EW // 16, _vec, 0)

    pltpu.sync_copy(g_l, shared.at[s])
    plsc.subcore_barrier()
    base = s * SL
    pltpu.sync_copy(shared.at[0, pl.ds(base, SL)], acc)
    for t in range(1, NS):
        pltpu.sync_copy(shared.at[t, pl.ds(base, SL)], tmp)

        def _mx(v, carry):
            sl = pl.ds(v * 16, 16)
            acc[sl] = jnp.maximum(acc[sl], tmp[sl])
            return carry
        lax.fori_loop(0, SL // 16, _mx, 0)
    pltpu.sync_copy(acc, g_out.at[pl.ds(c * NP + base, SL)])


_kb = pl.kernel(
    _kb_body,
    out_type=jax.ShapeDtypeStruct((NC * NP,), jnp.float32),
    mesh=_mesh,
    compiler_params=pltpu.CompilerParams(needs_layout_passes=False),
    scratch_types=[
        pltpu.VMEM((NP,), jnp.float32),
        pltpu.VMEM((NP,), jnp.float32),
        pltpu.VMEM((EW,), jnp.int32),
        pltpu.VMEM((EW,), jnp.int32),
        pltpu.VMEM((SL,), jnp.float32),
        pltpu.VMEM((SL,), jnp.float32),
        pltpu.VMEM_SHARED((NS, NP), jnp.float32),
    ],
)


def _kc_body(row3_hbm, col_hbm, f1_hbm, f2_hbm, gp_hbm, den_out, ex_out,
             f1_l, f2_l, m_l, gb, cbuf, rstage, exb, exfull, zb, den_sh):
    c = lax.axis_index("c")
    s = lax.axis_index("s")
    wid = s * NC + c
    pltpu.sync_copy(f1_hbm, f1_l)
    pltpu.sync_copy(f2_hbm, f2_l)
    pltpu.sync_copy(col_hbm.at[pl.ds(wid * EW, EW)], cbuf)
    pltpu.sync_copy(row3_hbm.at[wid], rstage)
    pltpu.sync_copy(gp_hbm.at[pl.ds(0, NP)], m_l)
    pltpu.sync_copy(gp_hbm.at[pl.ds(NP, NP)], gb)

    def _m(i, carry):
        sl = pl.ds(i * 16, 16)
        m_l[sl] = _lrelu(f1_l[sl] + jnp.maximum(m_l[sl], gb[sl]))
        return carry
    lax.fori_loop(0, NP // 16, _m, 0)

    def _z(i, carry):
        zb[pl.ds(i * 16, 16)] = jnp.zeros((16,), jnp.float32)
        return carry
    lax.fori_loop(0, SL // 16, _z, 0)
    pltpu.sync_copy(zb, den_sh.at[pl.ds(s * SL, SL)])
    plsc.subcore_barrier()

    def _row(j, carry):
        for v in range(VPS):
            sl = pl.ds(v * 16, 16)
            rows = rstage[j, sl]
            cols = cbuf[pl.ds(j * SUB + v * 16, 16)]
            x = plsc.load_gather(f1_l, [rows]) + plsc.load_gather(f2_l, [cols])
            mv = plsc.load_gather(m_l, [rows])
            ev = jnp.exp(_lrelu(x) - mv)
            exb[sl] = ev
            exfull[pl.ds(j * SUB + v * 16, 16)] = ev
        pltpu.sync_copy(exb, den_sh.at[rstage.at[j]], add=True)
        return carry
    lax.fori_loop(0, JR, _row, 0)
    pltpu.sync_copy(exfull, ex_out.at[pl.ds(wid * EW, EW)])
    plsc.subcore_barrier()
    sl = pl.ds(s * SL, SL)
    pltpu.sync_copy(den_sh.at[sl], den_out.at[pl.ds(c * NP + s * SL, SL)])


_kc = pl.kernel(
    _kc_body,
    out_type=(jax.ShapeDtypeStruct((NC * NP,), jnp.float32),
              jax.ShapeDtypeStruct((E,), jnp.float32)),
    mesh=_mesh,
    compiler_params=pltpu.CompilerParams(needs_layout_passes=False),
    scratch_types=[
        pltpu.VMEM((NP,), jnp.float32),
        pltpu.VMEM((NP,), jnp.float32),
        pltpu.VMEM((NP,), jnp.float32),
        pltpu.VMEM((NP,), jnp.float32),
        pltpu.VMEM((EW,), jnp.int32),
        pltpu.VMEM((JR, SUB), jnp.int32),
        pltpu.VMEM((SUB,), jnp.float32),
        pltpu.VMEM((EW,), jnp.float32),
        pltpu.VMEM((SL,), jnp.float32),
        pltpu.VMEM_SHARED((NP,), jnp.float32),
    ],
)


def _kd_body(row3_hbm, col_hbm, seq_hbm, ex_hbm, dp_hbm, vals_out,
             den_l, db, cbuf, rst, exb, fb, vals_sh,
             semg0, semg1, semx, semr, semsc0, semsc1):
    c = lax.axis_index("c")
    s = lax.axis_index("s")
    wid = s * NC + c
    pltpu.sync_copy(dp_hbm.at[pl.ds(0, N)], den_l)
    pltpu.sync_copy(col_hbm.at[pl.ds(wid * EW, EW)], cbuf)
    for k in range(5):
        pltpu.sync_copy(dp_hbm.at[pl.ds(NP + k * 2000, 2000)], db)

        def _add(i, carry):
            sl = pl.ds(k * 2000 + i * 16, 16)
            den_l[sl] = den_l[sl] + db[pl.ds(i * 16, 16)]
            return carry
        lax.fori_loop(0, 125, _add, 0)

    def _zf(e, carry):
        for v in range(D // 16):
            fb[0, e, pl.ds(v * 16, 16)] = jnp.zeros((16,), jnp.float32)
        return carry
    lax.fori_loop(0, SUB, _zf, 0)
    for k in range(SLV // SUB):
        pltpu.sync_copy(fb.at[0], vals_sh.at[pl.ds(s * SLV + k * SUB, SUB)])
    pltpu.sync_copy(fb.at[0, pl.ds(0, SLV % SUB)],
                    vals_sh.at[pl.ds(s * SLV + (SLV // SUB) * SUB,
                                     SLV % SUB)])

    pltpu.sync_copy(row3_hbm.at[wid, 0], rst.at[0])
    pltpu.sync_copy(row3_hbm.at[wid, 1], rst.at[1])
    pltpu.sync_copy(ex_hbm.at[pl.ds(wid * EW, SUB)], exb)
    pltpu.async_copy(seq_hbm.at[cbuf.at[pl.ds(0, SUB)]], fb.at[0], semg0)
    plsc.subcore_barrier()

    semg = (semg0, semg1)
    semsc = (semsc0, semsc1)

    def _coefs(b):
        cos = []
        for v in range(VPS):
            sl = pl.ds(v * 16, 16)
            dv = plsc.load_gather(den_l, [rst[b, sl]])
            cos.append(exb[sl] / dv)
        return cos

    def _scale(b, cos):
        for g in range(VPS):
            cv = cos[g]
            for l in range(16):
                cf = cv[l]
                e = g * 16 + l
                for v in range(D // 16):
                    fsl = pl.ds(v * 16, 16)
                    fb[b, e, fsl] = fb[b, e, fsl] * cf

    def _part(j, b, first):
        b1 = 1 - b
        cpr = None
        if not first:
            pltpu.make_async_copy(seq_hbm.at[pl.ds(0, SUB)], fb.at[b1],
                                  semsc[b1]).wait()
            cpr = pltpu.async_copy(row3_hbm.at[wid, j + 1], rst.at[b1],
                                   semr)
        pltpu.async_copy(seq_hbm.at[cbuf.at[pl.ds((j + 1) * SUB, SUB)]],
                         fb.at[b1], semg[b1])
        cos = _coefs(b)
        cpx = pltpu.async_copy(
            ex_hbm.at[pl.ds(wid * EW + (j + 1) * SUB, SUB)], exb, semx)
        pltpu.make_async_copy(seq_hbm.at[pl.ds(0, SUB)], fb.at[b],
                              semg[b]).wait()
        _scale(b, cos)
        pltpu.async_copy(fb.at[b], vals_sh.at[rst.at[b]], semsc[b],
                         add=True)
        cpx.wait()
        if cpr is not None:
            cpr.wait()

    _part(0, 0, True)
    _part(1, 1, False)

    def _dbl(t, carry):
        _part(2 * t, 0, False)
        _part(2 * t + 1, 1, False)
        return carry
    lax.fori_loop(1, JR // 2, _dbl, 0)

    pltpu.make_async_copy(seq_hbm.at[pl.ds(0, SUB)], fb.at[1],
                          semsc[1]).wait()
    cos = _coefs(0)
    pltpu.make_async_copy(seq_hbm.at[pl.ds(0, SUB)], fb.at[0], semg0).wait()
    _scale(0, cos)
    pltpu.sync_copy(fb.at[0], vals_sh.at[rst.at[0]], add=True)

    plsc.subcore_barrier()
    sl2 = pl.ds(s * SLV, SLV)
    pltpu.sync_copy(vals_sh.at[sl2], vals_out.at[c, sl2])


_kd = pl.kernel(
    _kd_body,
    out_type=jax.ShapeDtypeStruct((NC, NPV, D), jnp.float32),
    mesh=_mesh,
    compiler_params=pltpu.CompilerParams(needs_layout_passes=False),
    scratch_types=[
        pltpu.VMEM((N,), jnp.float32),
        pltpu.VMEM((2000,), jnp.float32),
        pltpu.VMEM((EW,), jnp.int32),
        pltpu.VMEM((2, SUB), jnp.int32),
        pltpu.VMEM((SUB,), jnp.float32),
        pltpu.VMEM((2, SUB, D), jnp.float32),
        pltpu.VMEM_SHARED((NPV, D), jnp.float32),
        pltpu.SemaphoreType.DMA,
        pltpu.SemaphoreType.DMA,
        pltpu.SemaphoreType.DMA,
        pltpu.SemaphoreType.DMA,
        pltpu.SemaphoreType.DMA,
        pltpu.SemaphoreType.DMA,
    ],
)


def _ke_body(v0_ref, v1_ref, b_ref, o_ref):
    o_ref[...] = v0_ref[0] + v1_ref[0] + b_ref[...]


_ke = pl.pallas_call(
    _ke_body,
    grid=(50,),
    in_specs=[
        pl.BlockSpec((1, 200, D), lambda i: (0, i, 0)),
        pl.BlockSpec((1, 200, D), lambda i: (1, i, 0)),
        pl.BlockSpec((1, D), lambda i: (0, 0)),
    ],
    out_specs=pl.BlockSpec((200, D), lambda i: (i, 0)),
    out_shape=jax.ShapeDtypeStruct((N, D), jnp.float32),
)


def kernel(feat, edge_index, W, a_l_w, a_l_b, a_r_w, a_r_b, bias):
    a_cat = jnp.concatenate(
        [a_l_w, a_r_w, jnp.zeros((D, D - 2), jnp.float32)], axis=1)
    seq, ff = _ka(feat, W, a_cat)
    f1 = jnp.pad(ff[:, 0] + a_l_b[0], (0, NP - N))
    f2 = jnp.pad(ff[:, 1] + a_r_b[0], (0, NP - N))
    row = edge_index[0]
    col = edge_index[1]
    row3 = row.reshape(NW, JR, SUB)
    g_p = _kb(row, col, f2)
    den_p, ex = _kc(row3, col, f1, f2, g_p)
    vals_p = _kd(row3, col, seq, ex, den_p)
    return _ke(vals_p, vals_p, bias.reshape(1, D))


# speedup vs baseline: 1.0121x; 1.0083x over previous
import jax
import jax.numpy as jnp
from jax import lax
from jax.experimental import pallas as pl
from jax.experimental.pallas import tpu as pltpu
from jax.experimental.pallas import tpu_sc as plsc

N = 10000
E = 320000
D = 128
NP = 10240
NC = 2
NS = 16
NW = NC * NS
EW = E // NW
SUB = 80
JR = EW // SUB
SL = NP // NS
NPV = 10112
SLV = NPV // NS
VPS = SUB // 16
NEG = -3.0e38

_mesh = plsc.VectorSubcoreMesh(
    core_axis_name="c", subcore_axis_name="s", num_cores=NC, num_subcores=NS)


def _lrelu(x):
    return jnp.maximum(x, 0.2 * x)


def _ka_body(f_ref, w_ref, a_ref, seq_ref, ff_ref):
    sft = jnp.dot(f_ref[...], w_ref[...], preferred_element_type=jnp.float32)
    seq_ref[...] = sft
    ff_ref[...] = jnp.dot(sft, a_ref[...], preferred_element_type=jnp.float32)


_ka = pl.pallas_call(
    _ka_body,
    grid=(50,),
    in_specs=[
        pl.BlockSpec((200, D), lambda i: (i, 0)),
        pl.BlockSpec((D, D), lambda i: (0, 0)),
        pl.BlockSpec((D, D), lambda i: (0, 0)),
    ],
    out_specs=[
        pl.BlockSpec((200, D), lambda i: (i, 0)),
        pl.BlockSpec((200, D), lambda i: (i, 0)),
    ],
    out_shape=[
        jax.ShapeDtypeStruct((N, D), jnp.float32),
        jax.ShapeDtypeStruct((N, D), jnp.float32),
    ],
)


def _kb_body(row_hbm, col_hbm, f2_hbm, g_out,
             f2_l, g_l, rbuf, cbuf, tmp, acc, shared):
    c = lax.axis_index("c")
    s = lax.axis_index("s")
    wid = s * NC + c
    pltpu.sync_copy(f2_hbm, f2_l)
    pltpu.sync_copy(row_hbm.at[pl.ds(wid * EW, EW)], rbuf)
    pltpu.sync_copy(col_hbm.at[pl.ds(wid * EW, EW)], cbuf)

    def _init(i, carry):
        g_l[pl.ds(i * 16, 16)] = jnp.full((16,), NEG, jnp.float32)
        return carry
    lax.fori_loop(0, NP // 16, _init, 0)

    def _vec(v, carry):
        sl = pl.ds(v * 16, 16)
        rows = rbuf[sl]
        cols = cbuf[sl]
        vals = plsc.load_gather(f2_l, [cols])

        def _cond(active):
            return jnp.any(active)

        def _body(active):
            cur = plsc.load_gather(g_l, [rows])
            need = jnp.logical_and(active, vals > cur)
            plsc.store_scatter(g_l, [rows], vals, mask=need)
            chk = plsc.load_gather(g_l, [rows])
            return jnp.logical_and(need, chk < vals)

        lax.while_loop(_cond, _body, jnp.ones((16,), jnp.bool_))
        return carry
    lax.fori_loop(0, EW // 16, _vec, 0)

    pltpu.sync_copy(g_l, shared.at[s])
    plsc.subcore_barrier()
    base = s * SL
    pltpu.sync_copy(shared.at[0, pl.ds(base, SL)], acc)
    for t in range(1, NS):
        pltpu.sync_copy(shared.at[t, pl.ds(base, SL)], tmp)

        def _mx(v, carry):
            sl = pl.ds(v * 16, 16)
            acc[sl] = jnp.maximum(acc[sl], tmp[sl])
            return carry
        lax.fori_loop(0, SL // 16, _mx, 0)
    pltpu.sync_copy(acc, g_out.at[pl.ds(c * NP + base, SL)])


_kb = pl.kernel(
    _kb_body,
    out_type=jax.ShapeDtypeStruct((NC * NP,), jnp.float32),
    mesh=_mesh,
    compiler_params=pltpu.CompilerParams(needs_layout_passes=False),
    scratch_types=[
        pltpu.VMEM((NP,), jnp.float32),
        pltpu.VMEM((NP,), jnp.float32),
        pltpu.VMEM((EW,), jnp.int32),
        pltpu.VMEM((EW,), jnp.int32),
        pltpu.VMEM((SL,), jnp.float32),
        pltpu.VMEM((SL,), jnp.float32),
        pltpu.VMEM_SHARED((NS, NP), jnp.float32),
    ],
)


def _kc_body(row3_hbm, col_hbm, f1_hbm, f2_hbm, gp_hbm, den_out, ex_out,
             f1_l, f2_l, m_l, gb, cbuf, rstage, exb, exfull, zb, den_sh):
    c = lax.axis_index("c")
    s = lax.axis_index("s")
    wid = s * NC + c
    pltpu.sync_copy(f1_hbm, f1_l)
    pltpu.sync_copy(f2_hbm, f2_l)
    pltpu.sync_copy(col_hbm.at[pl.ds(wid * EW, EW)], cbuf)
    pltpu.sync_copy(row3_hbm.at[wid], rstage)
    pltpu.sync_copy(gp_hbm.at[pl.ds(0, NP)], m_l)
    pltpu.sync_copy(gp_hbm.at[pl.ds(NP, NP)], gb)

    def _m(i, carry):
        sl = pl.ds(i * 16, 16)
        m_l[sl] = _lrelu(f1_l[sl] + jnp.maximum(m_l[sl], gb[sl]))
        return carry
    lax.fori_loop(0, NP // 16, _m, 0)

    def _z(i, carry):
        zb[pl.ds(i * 16, 16)] = jnp.zeros((16,), jnp.float32)
        return carry
    lax.fori_loop(0, SL // 16, _z, 0)
    pltpu.sync_copy(zb, den_sh.at[pl.ds(s * SL, SL)])
    plsc.subcore_barrier()

    def _row(j, carry):
        for v in range(VPS):
            sl = pl.ds(v * 16, 16)
            rows = rstage[j, sl]
            cols = cbuf[pl.ds(j * SUB + v * 16, 16)]
            x = plsc.load_gather(f1_l, [rows]) + plsc.load_gather(f2_l, [cols])
            mv = plsc.load_gather(m_l, [rows])
            ev = jnp.exp(_lrelu(x) - mv)
            exb[sl] = ev
            exfull[pl.ds(j * SUB + v * 16, 16)] = ev
        pltpu.sync_copy(exb, den_sh.at[rstage.at[j]], add=True)
        return carry
    lax.fori_loop(0, JR, _row, 0)
    pltpu.sync_copy(exfull, ex_out.at[pl.ds(wid * EW, EW)])
    plsc.subcore_barrier()
    sl = pl.ds(s * SL, SL)
    pltpu.sync_copy(den_sh.at[sl], den_out.at[pl.ds(c * NP + s * SL, SL)])


_kc = pl.kernel(
    _kc_body,
    out_type=(jax.ShapeDtypeStruct((NC * NP,), jnp.float32),
              jax.ShapeDtypeStruct((E,), jnp.float32)),
    mesh=_mesh,
    compiler_params=pltpu.CompilerParams(needs_layout_passes=False),
    scratch_types=[
        pltpu.VMEM((NP,), jnp.float32),
        pltpu.VMEM((NP,), jnp.float32),
        pltpu.VMEM((NP,), jnp.float32),
        pltpu.VMEM((NP,), jnp.float32),
        pltpu.VMEM((EW,), jnp.int32),
        pltpu.VMEM((JR, SUB), jnp.int32),
        pltpu.VMEM((SUB,), jnp.float32),
        pltpu.VMEM((EW,), jnp.float32),
        pltpu.VMEM((SL,), jnp.float32),
        pltpu.VMEM_SHARED((NP,), jnp.float32),
    ],
)


def _kf_body(dp_ref, o_ref):
    o_ref[...] = dp_ref[0] + dp_ref[1]


_kf = pl.pallas_call(
    _kf_body,
    grid=(1,),
    in_specs=[pl.BlockSpec((NC, 80, 128), lambda i: (0, 0, 0))],
    out_specs=pl.BlockSpec((80, 128), lambda i: (0, 0)),
    out_shape=jax.ShapeDtypeStruct((80, 128), jnp.float32),
)


def _kd_body(row3_hbm, col_hbm, seq_hbm, ex_hbm, den_hbm, vals_out,
             den_l, cbuf, rst, exb, fb, vals_sh,
             semg0, semg1, semx, semr, semsc0, semsc1):
    c = lax.axis_index("c")
    s = lax.axis_index("s")
    wid = s * NC + c
    pltpu.sync_copy(den_hbm.at[pl.ds(0, N)], den_l)
    pltpu.sync_copy(col_hbm.at[pl.ds(wid * EW, EW)], cbuf)

    def _zf(e, carry):
        for v in range(D // 16):
            fb[0, e, pl.ds(v * 16, 16)] = jnp.zeros((16,), jnp.float32)
        return carry
    lax.fori_loop(0, SUB, _zf, 0)
    for k in range(SLV // SUB):
        pltpu.sync_copy(fb.at[0], vals_sh.at[pl.ds(s * SLV + k * SUB, SUB)])
    pltpu.sync_copy(fb.at[0, pl.ds(0, SLV % SUB)],
                    vals_sh.at[pl.ds(s * SLV + (SLV // SUB) * SUB,
                                     SLV % SUB)])

    pltpu.sync_copy(row3_hbm.at[wid, 0], rst.at[0])
    pltpu.sync_copy(row3_hbm.at[wid, 1], rst.at[1])
    pltpu.sync_copy(ex_hbm.at[pl.ds(wid * EW, SUB)], exb)
    pltpu.async_copy(seq_hbm.at[cbuf.at[pl.ds(0, SUB)]], fb.at[0], semg0)
    plsc.subcore_barrier()

    semg = (semg0, semg1)
    semsc = (semsc0, semsc1)

    def _coefs(b):
        cos = []
        for v in range(VPS):
            sl = pl.ds(v * 16, 16)
            dv = plsc.load_gather(den_l, [rst[b, sl]])
            cos.append(exb[sl] / dv)
        return cos

    def _scale(b, cos):
        for g in range(VPS):
            cv = cos[g]
            for l in range(16):
                cf = cv[l]
                e = g * 16 + l
                for v in range(D // 16):
                    fsl = pl.ds(v * 16, 16)
                    fb[b, e, fsl] = fb[b, e, fsl] * cf

    def _part(j, b, first):
        b1 = 1 - b
        cpr = None
        if not first:
            pltpu.make_async_copy(seq_hbm.at[pl.ds(0, SUB)], fb.at[b1],
                                  semsc[b1]).wait()
            cpr = pltpu.async_copy(row3_hbm.at[wid, j + 1], rst.at[b1],
                                   semr)
        pltpu.async_copy(seq_hbm.at[cbuf.at[pl.ds((j + 1) * SUB, SUB)]],
                         fb.at[b1], semg[b1])
        cos = _coefs(b)
        cpx = pltpu.async_copy(
            ex_hbm.at[pl.ds(wid * EW + (j + 1) * SUB, SUB)], exb, semx)
        pltpu.make_async_copy(seq_hbm.at[pl.ds(0, SUB)], fb.at[b],
                              semg[b]).wait()
        _scale(b, cos)
        pltpu.async_copy(fb.at[b], vals_sh.at[rst.at[b]], semsc[b],
                         add=True)
        cpx.wait()
        if cpr is not None:
            cpr.wait()

    _part(0, 0, True)
    _part(1, 1, False)

    def _dbl(t, carry):
        _part(2 * t, 0, False)
        _part(2 * t + 1, 1, False)
        return carry
    lax.fori_loop(1, JR // 2, _dbl, 0)

    pltpu.make_async_copy(seq_hbm.at[pl.ds(0, SUB)], fb.at[1],
                          semsc[1]).wait()
    cos = _coefs(0)
    pltpu.make_async_copy(seq_hbm.at[pl.ds(0, SUB)], fb.at[0], semg0).wait()
    _scale(0, cos)
    pltpu.sync_copy(fb.at[0], vals_sh.at[rst.at[0]], add=True)

    plsc.subcore_barrier()
    sl2 = pl.ds(s * SLV, SLV)
    pltpu.sync_copy(vals_sh.at[sl2], vals_out.at[c, sl2])


_kd = pl.kernel(
    _kd_body,
    out_type=jax.ShapeDtypeStruct((NC, NPV, D), jnp.float32),
    mesh=_mesh,
    compiler_params=pltpu.CompilerParams(needs_layout_passes=False),
    scratch_types=[
        pltpu.VMEM((N,), jnp.float32),
        pltpu.VMEM((EW,), jnp.int32),
        pltpu.VMEM((2, SUB), jnp.int32),
        pltpu.VMEM((SUB,), jnp.float32),
        pltpu.VMEM((2, SUB, D), jnp.float32),
        pltpu.VMEM_SHARED((NPV, D), jnp.float32),
        pltpu.SemaphoreType.DMA,
        pltpu.SemaphoreType.DMA,
        pltpu.SemaphoreType.DMA,
        pltpu.SemaphoreType.DMA,
        pltpu.SemaphoreType.DMA,
        pltpu.SemaphoreType.DMA,
    ],
)


def _ke_body(v0_ref, v1_ref, b_ref, o_ref):
    o_ref[...] = v0_ref[0] + v1_ref[0] + b_ref[...]


_ke = pl.pallas_call(
    _ke_body,
    grid=(50,),
    in_specs=[
        pl.BlockSpec((1, 200, D), lambda i: (0, i, 0)),
        pl.BlockSpec((1, 200, D), lambda i: (1, i, 0)),
        pl.BlockSpec((1, D), lambda i: (0, 0)),
    ],
    out_specs=pl.BlockSpec((200, D), lambda i: (i, 0)),
    out_shape=jax.ShapeDtypeStruct((N, D), jnp.float32),
)


def kernel(feat, edge_index, W, a_l_w, a_l_b, a_r_w, a_r_b, bias):
    a_cat = jnp.concatenate(
        [a_l_w, a_r_w, jnp.zeros((D, D - 2), jnp.float32)], axis=1)
    seq, ff = _ka(feat, W, a_cat)
    f1 = jnp.pad(ff[:, 0] + a_l_b[0], (0, NP - N))
    f2 = jnp.pad(ff[:, 1] + a_r_b[0], (0, NP - N))
    row = edge_index[0]
    col = edge_index[1]
    row3 = row.reshape(NW, JR, SUB)
    g_p = _kb(row, col, f2)
    den_p, ex = _kc(row3, col, f1, f2, g_p)
    den = _kf(den_p.reshape(NC, 80, 128)).reshape(NC * NP // NC)
    vals_p = _kd(row3, col, seq, ex, den)
    return _ke(vals_p, vals_p, bias.reshape(1, D))
